# Initial kernel scaffold; baseline (speedup 1.0000x reference)
import functools

import jax
import jax.numpy as jnp
from jax import lax
from jax.experimental import pallas as pl
from jax.experimental.pallas import tpu as pltpu
from jax.experimental.pallas import tpu_sc as plsc

N = 10000
E = 160000
IN = 256
H = 512

NC = 2
NS = 16
L = 16

K = 128
E_SELF = E + N
PER_TILE = 10752
E_PAD = PER_TILE * NS
NB = PER_TILE // K
DEG_PER_TILE = E_PAD // (NC * NS)
DEG_VECS = DEG_PER_TILE // L
ACC_ROWS = 10048
DUMMY_ROW = ACC_ROWS - 1
ZSTRIPE = ACC_ROWS // NS
OSTRIPE = N // NS
MBLK = 1000
NQ = 4

_mesh = plsc.VectorSubcoreMesh(
    core_axis_name="c", subcore_axis_name="s", num_cores=NC, num_subcores=NS)


def _deg_body(cols_hbm, degpart_hbm, cbuf, hist, sem):
    c = lax.axis_index("c")
    s = lax.axis_index("s")
    wid = s * NC + c
    pltpu.async_copy(cols_hbm.at[wid], cbuf, sem).wait()
    zeros16 = jnp.zeros((L,), jnp.float32)
    ones16 = jnp.ones((L,), jnp.float32)

    @pl.loop(0, ACC_ROWS // L)
    def _zero(i):
        hist[pl.ds(i * L, L)] = zeros16

    @pl.loop(0, DEG_VECS)
    def _scat(i):
        idx = cbuf[i, :]
        plsc.addupdate_scatter(hist, [idx], ones16)

    pltpu.async_copy(hist, degpart_hbm.at[wid], sem).wait()


def _deg_call(cols3d):
    return pl.kernel(
        _deg_body,
        out_type=jax.ShapeDtypeStruct((NC * NS, ACC_ROWS), jnp.float32),
        mesh=_mesh,
        scratch_types=[
            pltpu.VMEM((DEG_VECS, L), jnp.int32),
            pltpu.VMEM((ACC_ROWS,), jnp.float32),
            pltpu.SemaphoreType.DMA,
        ],
    )(cols3d)


def _agg_body(y_hbm, rows_hbm, cols_hbm, out_hbm,
              rbuf, cbuf, gbuf, acc, gsem, ssem, csem):
    c = lax.axis_index("c")
    s = lax.axis_index("s")
    pltpu.async_copy(rows_hbm.at[s], rbuf, csem).wait()
    pltpu.async_copy(cols_hbm.at[s], cbuf, csem).wait()
    zeros16 = jnp.zeros((L,), jnp.float32)

    for p in range(2):
        q = c * 2 + p

        @pl.loop(0, K)
        def _zg(i):
            for jj in range(128 // L):
                gbuf[i, pl.ds(jj * L, L)] = zeros16

        for rb in range(4):
            pltpu.async_copy(
                gbuf, acc.at[pl.ds(s * ZSTRIPE + rb * K, K)], csem).wait()
        pltpu.async_copy(
            gbuf.at[pl.ds(0, ZSTRIPE - 4 * K)],
            acc.at[pl.ds(s * ZSTRIPE + 4 * K, ZSTRIPE - 4 * K)], csem).wait()
        plsc.subcore_barrier()

        @pl.loop(0, NB)
        def _edges(j):
            pltpu.async_copy(y_hbm.at[q].at[rbuf.at[j]], gbuf, gsem).wait()
            pltpu.async_copy(gbuf, acc.at[cbuf.at[j]], ssem, add=True).wait()

        plsc.subcore_barrier()
        pltpu.async_copy(
            acc.at[pl.ds(s * OSTRIPE, OSTRIPE)],
            out_hbm.at[q, pl.ds(s * OSTRIPE, OSTRIPE)], csem).wait()
        plsc.subcore_barrier()


def _agg_call(yq, rows3d, cols3d):
    return pl.kernel(
        _agg_body,
        out_type=jax.ShapeDtypeStruct((NQ, N, 128), jnp.float32),
        mesh=_mesh,
        scratch_types=[
            pltpu.VMEM((NB, K), jnp.int32),
            pltpu.VMEM((NB, K), jnp.int32),
            pltpu.VMEM((K, 128), jnp.float32),
            pltpu.VMEM_SHARED((ACC_ROWS, 128), jnp.float32),
            pltpu.SemaphoreType.DMA,
            pltpu.SemaphoreType.DMA,
            pltpu.SemaphoreType.DMA,
        ],
    )(yq, rows3d, cols3d)


def _dinv_body(degpart_ref, dinv_ref):
    deg = jnp.sum(degpart_ref[...], axis=0)[:N]
    dinv_ref[...] = lax.rsqrt(deg)[:, None]


def _dinv_call(degpart):
    return pl.pallas_call(
        _dinv_body,
        out_shape=jax.ShapeDtypeStruct((N, 1), jnp.float32),
    )(degpart)


def _mm1_body(x_ref, w_ref, dinv_ref, out_ref):
    y = dinv_ref[...] * jnp.dot(x_ref[...], w_ref[...],
                                preferred_element_type=jnp.float32)
    for qi in range(NQ):
        out_ref[qi] = y[:, qi * 128:(qi + 1) * 128]


def _mm1_call(x, w1, dinv):
    return pl.pallas_call(
        _mm1_body,
        grid=(N // MBLK,),
        in_specs=[
            pl.BlockSpec((MBLK, IN), lambda m: (m, 0)),
            pl.BlockSpec((IN, H), lambda m: (0, 0)),
            pl.BlockSpec((MBLK, 1), lambda m: (m, 0)),
        ],
        out_specs=pl.BlockSpec((NQ, MBLK, 128), lambda m: (0, m, 0)),
        out_shape=jax.ShapeDtypeStruct((NQ, N, 128), jnp.float32),
    )(x, w1, dinv)


def _mm2_body(agg_ref, w_ref, dinv_ref, b_ref, a_ref, out_ref):
    aggfull = jnp.concatenate([agg_ref[qi] for qi in range(NQ)], axis=1)
    dinv = dinv_ref[...]
    t = dinv * aggfull + b_ref[...]
    h = jnp.where(t > 0, t, a_ref[...] * t)
    y = dinv * jnp.dot(h, w_ref[...], preferred_element_type=jnp.float32)
    for qi in range(NQ):
        out_ref[qi] = y[:, qi * 128:(qi + 1) * 128]


def _mm2_call(agg, w2, dinv, b1, a1):
    return pl.pallas_call(
        _mm2_body,
        grid=(N // MBLK,),
        in_specs=[
            pl.BlockSpec((NQ, MBLK, 128), lambda m: (0, m, 0)),
            pl.BlockSpec((H, H), lambda m: (0, 0)),
            pl.BlockSpec((MBLK, 1), lambda m: (m, 0)),
            pl.BlockSpec((1, H), lambda m: (0, 0)),
            pl.BlockSpec((1, H), lambda m: (0, 0)),
        ],
        out_specs=pl.BlockSpec((NQ, MBLK, 128), lambda m: (0, m, 0)),
        out_shape=jax.ShapeDtypeStruct((NQ, N, 128), jnp.float32),
    )(agg, w2, dinv, b1, a1)


def _fin_body(agg_ref, dinv_ref, b_ref, a_ref, out_ref):
    aggfull = jnp.concatenate([agg_ref[qi] for qi in range(NQ)], axis=1)
    t = dinv_ref[...] * aggfull + b_ref[...]
    out_ref[...] = jnp.where(t > 0, t, a_ref[...] * t)


def _fin_call(agg, dinv, b2, a2):
    return pl.pallas_call(
        _fin_body,
        grid=(N // MBLK,),
        in_specs=[
            pl.BlockSpec((NQ, MBLK, 128), lambda m: (0, m, 0)),
            pl.BlockSpec((MBLK, 1), lambda m: (m, 0)),
            pl.BlockSpec((1, H), lambda m: (0, 0)),
            pl.BlockSpec((1, H), lambda m: (0, 0)),
        ],
        out_specs=pl.BlockSpec((MBLK, H), lambda m: (m, 0)),
        out_shape=jax.ShapeDtypeStruct((N, H), jnp.float32),
    )(agg, dinv, b2, a2)


def kernel(x, edge_index, W1, b1, a1, W2, b2, a2):
    ei = edge_index.astype(jnp.int32)
    loop = jnp.arange(N, dtype=jnp.int32)
    npad = E_PAD - E_SELF
    rows = jnp.concatenate([ei[0], loop, jnp.zeros((npad,), jnp.int32)])
    cols = jnp.concatenate([ei[1], loop,
                            jnp.full((npad,), DUMMY_ROW, jnp.int32)])
    rows3d = rows.reshape(NS, NB, K)
    cols3d = cols.reshape(NS, NB, K)
    colsdeg = cols.reshape(NC * NS, DEG_VECS, L)

    degpart = _deg_call(colsdeg)
    dinv = _dinv_call(degpart)

    y1 = _mm1_call(x, W1, dinv)
    agg1 = _agg_call(y1, rows3d, cols3d)
    y2 = _mm2_call(agg1, W2, dinv, b1.reshape(1, H), a1.reshape(1, H))
    agg2 = _agg_call(y2, rows3d, cols3d)
    return _fin_call(agg2, dinv, b2.reshape(1, H), a2.reshape(1, H))


# speedup vs baseline: 7.2439x; 7.2439x over previous
import functools

import jax
import jax.numpy as jnp
from jax import lax
from jax.experimental import pallas as pl
from jax.experimental.pallas import tpu as pltpu
from jax.experimental.pallas import tpu_sc as plsc

N = 10000
E = 160000
IN = 256
H = 512

NC = 2
NS = 16
L = 16

K = 128
E_SELF = E + N
PER_TILE = 10752
E_PAD = PER_TILE * NS
NB = PER_TILE // K
DEG_PER_TILE = E_PAD // (NC * NS)
DEG_VECS = DEG_PER_TILE // L
ACC_ROWS = 10112
DUMMY_ROW = ACC_ROWS - 1
ZSTRIPE = ACC_ROWS // NS
OSTRIPE = 624
MBLK = 1000
NQ = 4

@functools.cache
def _mesh():
    return plsc.VectorSubcoreMesh(
        core_axis_name="c", subcore_axis_name="s",
        num_cores=NC, num_subcores=NS)


_SC_PARAMS = pltpu.CompilerParams(needs_layout_passes=False)


def _deg_body(cols_hbm, degpart_hbm, cbuf, hist, sem):
    c = lax.axis_index("c")
    s = lax.axis_index("s")
    wid = s * NC + c
    pltpu.async_copy(cols_hbm.at[wid], cbuf, sem).wait()
    zeros16 = jnp.zeros((L,), jnp.float32)
    ones16 = jnp.ones((L,), jnp.float32)

    @pl.loop(0, ACC_ROWS // L)
    def _zero(i):
        hist[pl.ds(i * L, L)] = zeros16

    @pl.loop(0, DEG_VECS)
    def _scat(i):
        idx = cbuf[i, :]
        plsc.addupdate_scatter(hist, [idx], ones16)

    pltpu.async_copy(hist, degpart_hbm.at[wid, 0], sem).wait()


def _deg_call(cols3d):
    return pl.kernel(
        _deg_body,
        out_type=jax.ShapeDtypeStruct((NC * NS, 1, ACC_ROWS), jnp.float32),
        mesh=_mesh(),
        compiler_params=_SC_PARAMS,
        scratch_types=[
            pltpu.VMEM((DEG_VECS, L), jnp.int32),
            pltpu.VMEM((ACC_ROWS,), jnp.float32),
            pltpu.SemaphoreType.DMA,
        ],
    )(cols3d)


def _agg_body(y_hbm, rows_hbm, cols_hbm, out_hbm,
              rbuf, cbuf, gbuf, acc, gsem, ssem, csem):
    c = lax.axis_index("c")
    s = lax.axis_index("s")
    pltpu.async_copy(rows_hbm.at[s], rbuf, csem).wait()
    pltpu.async_copy(cols_hbm.at[s], cbuf, csem).wait()
    zeros16 = jnp.zeros((L,), jnp.float32)

    for p in range(2):
        q = c * 2 + p

        @pl.loop(0, K)
        def _zg(i):
            for jj in range(128 // L):
                gbuf[i, pl.ds(jj * L, L)] = zeros16

        for rb in range(4):
            pltpu.async_copy(
                gbuf, acc.at[pl.ds(s * ZSTRIPE + rb * K, K)], csem).wait()
        pltpu.async_copy(
            gbuf.at[pl.ds(0, ZSTRIPE - 4 * K)],
            acc.at[pl.ds(s * ZSTRIPE + 4 * K, ZSTRIPE - 4 * K)], csem).wait()
        plsc.subcore_barrier()

        @pl.loop(0, NB)
        def _edges(j):
            pltpu.async_copy(y_hbm.at[q].at[rbuf.at[j]], gbuf, gsem).wait()
            pltpu.async_copy(gbuf, acc.at[cbuf.at[j]], ssem, add=True).wait()

        plsc.subcore_barrier()
        pltpu.async_copy(
            acc.at[pl.ds(s * OSTRIPE, OSTRIPE)],
            out_hbm.at[q, pl.ds(s * OSTRIPE, OSTRIPE)], csem).wait()

        @pl.when(s == 0)
        def _tail():
            pltpu.async_copy(
                acc.at[pl.ds(NS * OSTRIPE, N - NS * OSTRIPE)],
                out_hbm.at[q, pl.ds(NS * OSTRIPE, N - NS * OSTRIPE)],
                csem).wait()

        plsc.subcore_barrier()


def _agg_call(yq, rows3d, cols3d):
    return pl.kernel(
        _agg_body,
        out_type=jax.ShapeDtypeStruct((NQ, N, 128), jnp.float32),
        mesh=_mesh(),
        compiler_params=_SC_PARAMS,
        scratch_types=[
            pltpu.VMEM((NB, K), jnp.int32),
            pltpu.VMEM((NB, K), jnp.int32),
            pltpu.VMEM((K, 128), jnp.float32),
            pltpu.VMEM_SHARED((ACC_ROWS, 128), jnp.float32),
            pltpu.SemaphoreType.DMA,
            pltpu.SemaphoreType.DMA,
            pltpu.SemaphoreType.DMA,
        ],
    )(yq, rows3d, cols3d)


def _dinv_body(degpart_ref, dinv_ref):
    deg = jnp.sum(degpart_ref[...], axis=0)[:N]
    dinv_ref[...] = lax.rsqrt(deg)[:, None]


def _dinv_call(degpart):
    return pl.pallas_call(
        _dinv_body,
        out_shape=jax.ShapeDtypeStruct((N, 1), jnp.float32),
    )(degpart)


def _mm1_body(x_ref, w_ref, dinv_ref, out_ref):
    y = dinv_ref[...] * jnp.dot(x_ref[...], w_ref[...],
                                preferred_element_type=jnp.float32)
    for qi in range(NQ):
        out_ref[qi] = y[:, qi * 128:(qi + 1) * 128]


def _mm1_call(x, w1, dinv):
    return pl.pallas_call(
        _mm1_body,
        grid=(N // MBLK,),
        in_specs=[
            pl.BlockSpec((MBLK, IN), lambda m: (m, 0)),
            pl.BlockSpec((IN, H), lambda m: (0, 0)),
            pl.BlockSpec((MBLK, 1), lambda m: (m, 0)),
        ],
        out_specs=pl.BlockSpec((NQ, MBLK, 128), lambda m: (0, m, 0)),
        out_shape=jax.ShapeDtypeStruct((NQ, N, 128), jnp.float32),
    )(x, w1, dinv)


def _mm2_body(agg_ref, w_ref, dinv_ref, b_ref, a_ref, out_ref):
    aggfull = jnp.concatenate([agg_ref[qi] for qi in range(NQ)], axis=1)
    dinv = dinv_ref[...]
    t = dinv * aggfull + b_ref[...]
    h = jnp.where(t > 0, t, a_ref[...] * t)
    y = dinv * jnp.dot(h, w_ref[...], preferred_element_type=jnp.float32)
    for qi in range(NQ):
        out_ref[qi] = y[:, qi * 128:(qi + 1) * 128]


def _mm2_call(agg, w2, dinv, b1, a1):
    return pl.pallas_call(
        _mm2_body,
        grid=(N // MBLK,),
        in_specs=[
            pl.BlockSpec((NQ, MBLK, 128), lambda m: (0, m, 0)),
            pl.BlockSpec((H, H), lambda m: (0, 0)),
            pl.BlockSpec((MBLK, 1), lambda m: (m, 0)),
            pl.BlockSpec((1, H), lambda m: (0, 0)),
            pl.BlockSpec((1, H), lambda m: (0, 0)),
        ],
        out_specs=pl.BlockSpec((NQ, MBLK, 128), lambda m: (0, m, 0)),
        out_shape=jax.ShapeDtypeStruct((NQ, N, 128), jnp.float32),
    )(agg, w2, dinv, b1, a1)


def _fin_body(agg_ref, dinv_ref, b_ref, a_ref, out_ref):
    aggfull = jnp.concatenate([agg_ref[qi] for qi in range(NQ)], axis=1)
    t = dinv_ref[...] * aggfull + b_ref[...]
    out_ref[...] = jnp.where(t > 0, t, a_ref[...] * t)


def _fin_call(agg, dinv, b2, a2):
    return pl.pallas_call(
        _fin_body,
        grid=(N // MBLK,),
        in_specs=[
            pl.BlockSpec((NQ, MBLK, 128), lambda m: (0, m, 0)),
            pl.BlockSpec((MBLK, 1), lambda m: (m, 0)),
            pl.BlockSpec((1, H), lambda m: (0, 0)),
            pl.BlockSpec((1, H), lambda m: (0, 0)),
        ],
        out_specs=pl.BlockSpec((MBLK, H), lambda m: (m, 0)),
        out_shape=jax.ShapeDtypeStruct((N, H), jnp.float32),
    )(agg, dinv, b2, a2)


def kernel(x, edge_index, W1, b1, a1, W2, b2, a2):
    ei = edge_index.astype(jnp.int32)
    loop = jnp.arange(N, dtype=jnp.int32)
    npad = E_PAD - E_SELF
    rows = jnp.concatenate([ei[0], loop, jnp.zeros((npad,), jnp.int32)])
    cols = jnp.concatenate([ei[1], loop,
                            jnp.full((npad,), DUMMY_ROW, jnp.int32)])
    rows3d = rows.reshape(NS, NB, K)
    cols3d = cols.reshape(NS, NB, K)
    colsdeg = cols.reshape(NC * NS, DEG_VECS, L)

    degpart = _deg_call(colsdeg)
    dinv = _dinv_call(degpart.reshape(NC * NS, ACC_ROWS))

    y1 = _mm1_call(x, W1, dinv)
    agg1 = _agg_call(y1, rows3d, cols3d)
    y2 = _mm2_call(agg1, W2, dinv, b1.reshape(1, H), a1.reshape(1, H))
    agg2 = _agg_call(y2, rows3d, cols3d)
    return _fin_call(agg2, dinv, b2.reshape(1, H), a2.reshape(1, H))


# speedup vs baseline: 9.2963x; 1.2833x over previous
import functools

import jax
import jax.numpy as jnp
from jax import lax
from jax.experimental import pallas as pl
from jax.experimental.pallas import tpu as pltpu
from jax.experimental.pallas import tpu_sc as plsc

N = 10000
E = 160000
IN = 256
H = 512

NC = 2
NS = 16
L = 16

K = 96
E_SELF = E + N
PER_TILE = 10752
E_PAD = PER_TILE * NS
NB = PER_TILE // K
DEG_PER_TILE = E_PAD // (NC * NS)
DEG_VECS = DEG_PER_TILE // L
ACC_ROWS = 10112
DUMMY_ROW = ACC_ROWS - 1
ZSTRIPE = ACC_ROWS // NS
OSTRIPE = 624
MBLK = 1000
NQ = 4

@functools.cache
def _mesh():
    return plsc.VectorSubcoreMesh(
        core_axis_name="c", subcore_axis_name="s",
        num_cores=NC, num_subcores=NS)


_SC_PARAMS = pltpu.CompilerParams(needs_layout_passes=False)


def _deg_body(cols_hbm, degpart_hbm, cbuf, hist, sem):
    c = lax.axis_index("c")
    s = lax.axis_index("s")
    wid = s * NC + c
    pltpu.async_copy(cols_hbm.at[wid], cbuf, sem).wait()
    zeros16 = jnp.zeros((L,), jnp.float32)
    ones16 = jnp.ones((L,), jnp.float32)

    @pl.loop(0, ACC_ROWS // L)
    def _zero(i):
        hist[pl.ds(i * L, L)] = zeros16

    @pl.loop(0, DEG_VECS)
    def _scat(i):
        idx = cbuf[i, :]
        plsc.addupdate_scatter(hist, [idx], ones16)

    pltpu.async_copy(hist, degpart_hbm.at[wid, 0], sem).wait()


def _deg_call(cols3d):
    return pl.kernel(
        _deg_body,
        out_type=jax.ShapeDtypeStruct((NC * NS, 1, ACC_ROWS), jnp.float32),
        mesh=_mesh(),
        compiler_params=_SC_PARAMS,
        scratch_types=[
            pltpu.VMEM((DEG_VECS, L), jnp.int32),
            pltpu.VMEM((ACC_ROWS,), jnp.float32),
            pltpu.SemaphoreType.DMA,
        ],
    )(cols3d)


def _agg_body(y_hbm, rows_hbm, cols_hbm, out_hbm,
              rbuf, cbuf, gbuf0, gbuf1, acc, gsem0, gsem1, ssem0, ssem1,
              csem):
    c = lax.axis_index("c")
    s = lax.axis_index("s")
    pltpu.async_copy(rows_hbm.at[s, 0], rbuf, csem).wait()
    pltpu.async_copy(cols_hbm.at[s], cbuf, csem).wait()
    zeros16 = jnp.zeros((L,), jnp.float32)

    for p in range(2):
        q = c * 2 + p

        @pl.loop(0, K)
        def _zg(i):
            for jj in range(128 // L):
                gbuf0[i, pl.ds(jj * L, L)] = zeros16

        for rb in range(ZSTRIPE // K):
            pltpu.async_copy(
                gbuf0, acc.at[pl.ds(s * ZSTRIPE + rb * K, K)], csem).wait()
        _zrem = ZSTRIPE % K
        if _zrem:
            pltpu.async_copy(
                gbuf0.at[pl.ds(0, _zrem)],
                acc.at[pl.ds(s * ZSTRIPE + ZSTRIPE - _zrem, _zrem)],
                csem).wait()
        plsc.subcore_barrier()

        def _ridx(j):
            return rbuf.at[pl.ds(j * K, K)]

        pltpu.async_copy(y_hbm.at[q].at[_ridx(0)], gbuf0, gsem0)

        @pl.loop(0, NB // 2)
        def _edges(k):
            j0 = 2 * k
            pltpu.async_copy(y_hbm.at[q].at[_ridx(j0 + 1)], gbuf1, gsem1)
            pltpu.make_async_copy(y_hbm.at[q].at[_ridx(j0)], gbuf0,
                                  gsem0).wait()
            pltpu.async_copy(gbuf0, acc.at[cbuf.at[j0]], ssem0,
                             add=True).wait()

            @pl.when(k < NB // 2 - 1)
            def _pref():
                pltpu.async_copy(y_hbm.at[q].at[_ridx(j0 + 2)], gbuf0,
                                 gsem0)

            pltpu.make_async_copy(y_hbm.at[q].at[_ridx(j0 + 1)], gbuf1,
                                  gsem1).wait()
            pltpu.async_copy(gbuf1, acc.at[cbuf.at[j0 + 1]], ssem1,
                             add=True).wait()

        plsc.subcore_barrier()
        pltpu.async_copy(
            acc.at[pl.ds(s * OSTRIPE, OSTRIPE)],
            out_hbm.at[q, pl.ds(s * OSTRIPE, OSTRIPE)], csem).wait()

        @pl.when(s == 0)
        def _tail():
            pltpu.async_copy(
                acc.at[pl.ds(NS * OSTRIPE, N - NS * OSTRIPE)],
                out_hbm.at[q, pl.ds(NS * OSTRIPE, N - NS * OSTRIPE)],
                csem).wait()

        plsc.subcore_barrier()


def _agg_call(yq, rows3d, cols3d):
    return pl.kernel(
        _agg_body,
        out_type=jax.ShapeDtypeStruct((NQ, N, 128), jnp.float32),
        mesh=_mesh(),
        compiler_params=_SC_PARAMS,
        scratch_types=[
            pltpu.VMEM((PER_TILE,), jnp.int32),
            pltpu.VMEM((NB, K), jnp.int32),
            pltpu.VMEM((K, 128), jnp.float32),
            pltpu.VMEM((K, 128), jnp.float32),
            pltpu.VMEM_SHARED((ACC_ROWS, 128), jnp.float32),
            pltpu.SemaphoreType.DMA,
            pltpu.SemaphoreType.DMA,
            pltpu.SemaphoreType.DMA,
            pltpu.SemaphoreType.DMA,
            pltpu.SemaphoreType.DMA,
        ],
    )(yq, rows3d, cols3d)


def _dinv_body(degpart_ref, dinv_ref):
    deg = jnp.sum(degpart_ref[...], axis=0)[:N]
    dinv_ref[...] = lax.rsqrt(deg)[:, None]


def _dinv_call(degpart):
    return pl.pallas_call(
        _dinv_body,
        out_shape=jax.ShapeDtypeStruct((N, 1), jnp.float32),
    )(degpart)


def _mm1_body(x_ref, w_ref, dinv_ref, out_ref):
    y = dinv_ref[...] * jnp.dot(x_ref[...], w_ref[...],
                                preferred_element_type=jnp.float32)
    for qi in range(NQ):
        out_ref[qi] = y[:, qi * 128:(qi + 1) * 128]


def _mm1_call(x, w1, dinv):
    return pl.pallas_call(
        _mm1_body,
        grid=(N // MBLK,),
        in_specs=[
            pl.BlockSpec((MBLK, IN), lambda m: (m, 0)),
            pl.BlockSpec((IN, H), lambda m: (0, 0)),
            pl.BlockSpec((MBLK, 1), lambda m: (m, 0)),
        ],
        out_specs=pl.BlockSpec((NQ, MBLK, 128), lambda m: (0, m, 0)),
        out_shape=jax.ShapeDtypeStruct((NQ, N, 128), jnp.float32),
    )(x, w1, dinv)


def _mm2_body(agg_ref, w_ref, dinv_ref, b_ref, a_ref, out_ref):
    aggfull = jnp.concatenate([agg_ref[qi] for qi in range(NQ)], axis=1)
    dinv = dinv_ref[...]
    t = dinv * aggfull + b_ref[...]
    h = jnp.where(t > 0, t, a_ref[...] * t)
    y = dinv * jnp.dot(h, w_ref[...], preferred_element_type=jnp.float32)
    for qi in range(NQ):
        out_ref[qi] = y[:, qi * 128:(qi + 1) * 128]


def _mm2_call(agg, w2, dinv, b1, a1):
    return pl.pallas_call(
        _mm2_body,
        grid=(N // MBLK,),
        in_specs=[
            pl.BlockSpec((NQ, MBLK, 128), lambda m: (0, m, 0)),
            pl.BlockSpec((H, H), lambda m: (0, 0)),
            pl.BlockSpec((MBLK, 1), lambda m: (m, 0)),
            pl.BlockSpec((1, H), lambda m: (0, 0)),
            pl.BlockSpec((1, H), lambda m: (0, 0)),
        ],
        out_specs=pl.BlockSpec((NQ, MBLK, 128), lambda m: (0, m, 0)),
        out_shape=jax.ShapeDtypeStruct((NQ, N, 128), jnp.float32),
    )(agg, w2, dinv, b1, a1)


def _fin_body(agg_ref, dinv_ref, b_ref, a_ref, out_ref):
    aggfull = jnp.concatenate([agg_ref[qi] for qi in range(NQ)], axis=1)
    t = dinv_ref[...] * aggfull + b_ref[...]
    out_ref[...] = jnp.where(t > 0, t, a_ref[...] * t)


def _fin_call(agg, dinv, b2, a2):
    return pl.pallas_call(
        _fin_body,
        grid=(N // MBLK,),
        in_specs=[
            pl.BlockSpec((NQ, MBLK, 128), lambda m: (0, m, 0)),
            pl.BlockSpec((MBLK, 1), lambda m: (m, 0)),
            pl.BlockSpec((1, H), lambda m: (0, 0)),
            pl.BlockSpec((1, H), lambda m: (0, 0)),
        ],
        out_specs=pl.BlockSpec((MBLK, H), lambda m: (m, 0)),
        out_shape=jax.ShapeDtypeStruct((N, H), jnp.float32),
    )(agg, dinv, b2, a2)


def kernel(x, edge_index, W1, b1, a1, W2, b2, a2):
    ei = edge_index.astype(jnp.int32)
    loop = jnp.arange(N, dtype=jnp.int32)
    npad = E_PAD - E_SELF
    rows = jnp.concatenate([ei[0], loop, jnp.zeros((npad,), jnp.int32)])
    cols = jnp.concatenate([ei[1], loop,
                            jnp.full((npad,), DUMMY_ROW, jnp.int32)])
    rows3d = rows.reshape(NS, 1, PER_TILE)
    cols3d = cols.reshape(NS, NB, K)
    colsdeg = cols.reshape(NC * NS, DEG_VECS, L)

    degpart = _deg_call(colsdeg)
    dinv = _dinv_call(degpart.reshape(NC * NS, ACC_ROWS))

    y1 = _mm1_call(x, W1, dinv)
    agg1 = _agg_call(y1, rows3d, cols3d)
    y2 = _mm2_call(agg1, W2, dinv, b1.reshape(1, H), a1.reshape(1, H))
    agg2 = _agg_call(y2, rows3d, cols3d)
    return _fin_call(agg2, dinv, b2.reshape(1, H), a2.reshape(1, H))
